# Initial kernel scaffold; baseline (speedup 1.0000x reference)
import jax
import jax.numpy as jnp
from jax.experimental import pallas as pl


def kernel(hash_out, cls_out, target, ind, target_vectors, U, Y):
    raise NotImplementedError("write your pallas kernel here")


# speedup vs baseline: 3.2699x; 3.2699x over previous
import jax
import jax.numpy as jnp
from jax.experimental import pallas as pl
from jax.experimental.pallas import tpu as pltpu

_B = 4096
_NC = 100
_HB = 64
_M = 16.0
_ALPHA = 0.1
_BETA = 0.1


def _loss_kernel(hash_ref, cls_ref, target_ref, tv_ref, out_ref):
    tgt = target_ref[...]
    col = jax.lax.broadcasted_iota(jnp.int32, (_B, _NC), 1)
    row_max = jnp.max(tgt, axis=1, keepdims=True)
    label = jnp.min(jnp.where(tgt == row_max, col, _NC), axis=1, keepdims=True)
    onehot = (col == label).astype(jnp.float32)

    def ce(logits):
        m = jnp.max(logits, axis=1, keepdims=True)
        lse = m[:, 0] + jnp.log(jnp.sum(jnp.exp(logits - m), axis=1))
        picked = jnp.sum(onehot * logits, axis=1)
        return -jnp.mean(picked - lse)

    cls_loss = 0.5 * ce(cls_ref[0]) + 0.5 * ce(cls_ref[1])

    t = jnp.dot(onehot.astype(jnp.bfloat16), tv_ref[...].astype(jnp.bfloat16),
                preferred_element_type=jnp.float32)

    h0 = hash_ref[0]
    h1 = hash_ref[1]
    pol0 = jnp.mean(jnp.maximum(_M - h0 * t, 0.0))
    pol1 = jnp.mean(jnp.maximum(_M - h1 * t, 0.0))

    neg = jnp.sum((h0 < 0).astype(jnp.float32)) + jnp.sum((h1 < 0).astype(jnp.float32))
    pos = jnp.sum((h0 > 0).astype(jnp.float32)) + jnp.sum((h1 > 0).astype(jnp.float32))
    denom = 2.0 * (2 * _HB) * _B
    p_m1 = neg / denom
    p_1 = pos / denom
    inv_ln2 = 1.4426950408889634
    b_loss = jnp.abs(-p_m1 * jnp.log(p_m1) * inv_ln2 + p_1 * jnp.log(p_1) * inv_ln2)

    out_ref[0] = cls_loss + _ALPHA * (pol0 + pol1) + _BETA * b_loss


def kernel(hash_out, cls_out, target, ind, target_vectors, U, Y):
    out = pl.pallas_call(
        _loss_kernel,
        out_shape=jax.ShapeDtypeStruct((1,), jnp.float32),
        out_specs=pl.BlockSpec(memory_space=pltpu.SMEM),
    )(hash_out, cls_out, target, target_vectors)
    return out[0]
